# baseline (speedup 1.0000x reference)
import jax
import jax.numpy as jnp
from jax.experimental import pallas as pl

_OUT = 65
_L = 2048
_H = 128
_MROWS = 72
_CB = 8


def _fused_kernel(x_ref, m_ref, mask_ref, out_ref):
    xb = x_ref[0]
    nz = xb != 0.0
    idx = jax.lax.broadcasted_iota(jnp.int32, xb.shape, 1)
    first = jnp.min(jnp.where(nz, idx, _L), axis=1, keepdims=True)
    last = jnp.max(jnp.where(nz, idx, -1), axis=1, keepdims=True)
    mask_ref[0] = (idx >= first) & (idx <= last)
    out_ref[0] = m_ref[0, :, :_OUT, :_OUT]


def kernel(x, matrix):
    N, C, L = x.shape
    mask, resized = pl.pallas_call(
        _fused_kernel,
        grid=(N, C // _CB),
        in_specs=[
            pl.BlockSpec((1, _CB, L), lambda n, c: (n, c, 0)),
            pl.BlockSpec((1, _CB, _MROWS, _H), lambda n, c: (n, c, 0, 0)),
        ],
        out_specs=[
            pl.BlockSpec((1, _CB, L), lambda n, c: (n, c, 0)),
            pl.BlockSpec((1, _CB, _OUT, _OUT), lambda n, c: (n, c, 0, 0)),
        ],
        out_shape=[
            jax.ShapeDtypeStruct((N, C, L), jnp.bool_),
            jax.ShapeDtypeStruct((N, C, _OUT, _OUT), jnp.float32),
        ],
    )(x, matrix)
    return mask, resized


# speedup vs baseline: 1.6133x; 1.6133x over previous
import jax
import jax.numpy as jnp
from jax.experimental import pallas as pl

_OUT = 65
_L = 2048
_H = 128
_MROWS = 72
_CB = 32


def _fused_kernel(x_ref, m_ref, mask_ref, out_ref):
    xb = x_ref[0]
    nz = xb != 0.0
    idx = jax.lax.broadcasted_iota(jnp.int32, xb.shape, 1)
    first = jnp.min(jnp.where(nz, idx, _L), axis=1, keepdims=True)
    last = jnp.max(jnp.where(nz, idx, -1), axis=1, keepdims=True)
    mask_ref[0] = (idx >= first) & (idx <= last)
    out_ref[0] = m_ref[0, :, :_OUT, :_OUT]


def kernel(x, matrix):
    N, C, L = x.shape
    mask, resized = pl.pallas_call(
        _fused_kernel,
        grid=(N, C // _CB),
        in_specs=[
            pl.BlockSpec((1, _CB, L), lambda n, c: (n, c, 0)),
            pl.BlockSpec((1, _CB, _MROWS, _H), lambda n, c: (n, c, 0, 0)),
        ],
        out_specs=[
            pl.BlockSpec((1, _CB, L), lambda n, c: (n, c, 0)),
            pl.BlockSpec((1, _CB, _OUT, _OUT), lambda n, c: (n, c, 0, 0)),
        ],
        out_shape=[
            jax.ShapeDtypeStruct((N, C, L), jnp.bool_),
            jax.ShapeDtypeStruct((N, C, _OUT, _OUT), jnp.float32),
        ],
    )(x, matrix)
    return mask, resized


# speedup vs baseline: 1.7803x; 1.1035x over previous
import jax
import jax.numpy as jnp
from jax.experimental import pallas as pl

_OUT = 65
_L = 2048
_H = 128
_MROWS = 72
_CB = 32
_NB = 2


def _fused_kernel(x_ref, m_ref, mask_ref, out_ref):
    xb = x_ref[...]
    nz = xb != 0.0
    idx = jax.lax.broadcasted_iota(jnp.int32, xb.shape, 2)
    first = jnp.min(jnp.where(nz, idx, _L), axis=2, keepdims=True)
    last = jnp.max(jnp.where(nz, idx, -1), axis=2, keepdims=True)
    mask_ref[...] = (idx >= first) & (idx <= last)
    out_ref[...] = m_ref[:, :, :_OUT, :_OUT]


def kernel(x, matrix):
    N, C, L = x.shape
    mask, resized = pl.pallas_call(
        _fused_kernel,
        grid=(N // _NB, C // _CB),
        in_specs=[
            pl.BlockSpec((_NB, _CB, L), lambda n, c: (n, c, 0)),
            pl.BlockSpec((_NB, _CB, _MROWS, _H), lambda n, c: (n, c, 0, 0)),
        ],
        out_specs=[
            pl.BlockSpec((_NB, _CB, L), lambda n, c: (n, c, 0)),
            pl.BlockSpec((_NB, _CB, _OUT, _OUT), lambda n, c: (n, c, 0, 0)),
        ],
        out_shape=[
            jax.ShapeDtypeStruct((N, C, L), jnp.bool_),
            jax.ShapeDtypeStruct((N, C, _OUT, _OUT), jnp.float32),
        ],
    )(x, matrix)
    return mask, resized


# speedup vs baseline: 1.8472x; 1.0376x over previous
import jax
import jax.numpy as jnp
from jax.experimental import pallas as pl

_OUT = 65
_L = 2048
_H = 128
_MROWS = 72
_CB = 32
_NB = 4


def _fused_kernel(x_ref, m_ref, mask_ref, out_ref):
    xb = x_ref[...]
    nz = xb != 0.0
    idx = jax.lax.broadcasted_iota(jnp.int32, xb.shape, 2)
    first = jnp.min(jnp.where(nz, idx, _L), axis=2, keepdims=True)
    last = jnp.max(jnp.where(nz, idx, -1), axis=2, keepdims=True)
    mask_ref[...] = (idx >= first) & (idx <= last)
    out_ref[...] = m_ref[:, :, :_OUT, :_OUT]


def kernel(x, matrix):
    N, C, L = x.shape
    mask, resized = pl.pallas_call(
        _fused_kernel,
        grid=(N // _NB, C // _CB),
        in_specs=[
            pl.BlockSpec((_NB, _CB, L), lambda n, c: (n, c, 0)),
            pl.BlockSpec((_NB, _CB, _MROWS, _H), lambda n, c: (n, c, 0, 0)),
        ],
        out_specs=[
            pl.BlockSpec((_NB, _CB, L), lambda n, c: (n, c, 0)),
            pl.BlockSpec((_NB, _CB, _OUT, _OUT), lambda n, c: (n, c, 0, 0)),
        ],
        out_shape=[
            jax.ShapeDtypeStruct((N, C, L), jnp.bool_),
            jax.ShapeDtypeStruct((N, C, _OUT, _OUT), jnp.float32),
        ],
    )(x, matrix)
    return mask, resized
